# Initial kernel scaffold; baseline (speedup 1.0000x reference)
import functools

import jax
import jax.numpy as jnp
from jax.experimental import pallas as pl
from jax.experimental.pallas import tpu as pltpu

HIDDEN = 2048
INTER = 1408
N_ROUTED = 8
TOP_K = 2
LANES = 128


def _router_body(x_ref, gwt_ref, bias_ref, idx_ref, w_ref):
    xb = x_ref[...]
    logits = jax.lax.dot_general(
        xb, gwt_ref[...], (((1,), (0,)), ((), ())),
        preferred_element_type=jnp.float32,
        precision=jax.lax.Precision.HIGHEST,
    )
    scores = jax.nn.sigmoid(logits)
    bm = scores.shape[0]
    lane = jax.lax.broadcasted_iota(jnp.int32, (bm, LANES), 1)
    valid = lane < N_ROUTED
    tie = lane.astype(jnp.float32) * 1e-6
    neg = jnp.float32(-1e30)
    sr = jnp.where(valid, scores + bias_ref[...] + tie, neg)
    m1 = jnp.max(sr, axis=1, keepdims=True)
    i1 = jnp.min(jnp.where(sr == m1, lane, LANES), axis=1, keepdims=True)
    sr2 = jnp.where(lane == i1, neg, sr)
    m2 = jnp.max(sr2, axis=1, keepdims=True)
    i2 = jnp.min(jnp.where(sr2 == m2, lane, LANES), axis=1, keepdims=True)
    s1 = jnp.sum(jnp.where(lane == i1, scores, 0.0), axis=1, keepdims=True)
    s2 = jnp.sum(jnp.where(lane == i2, scores, 0.0), axis=1, keepdims=True)
    denom = jnp.maximum(s1 + s2, 1e-9)
    idx_ref[...] = jnp.where(lane == 0, i1, jnp.where(lane == 1, i2, 0))
    w_ref[...] = jnp.where(lane == 0, s1 / denom,
                           jnp.where(lane == 1, s2 / denom, 0.0))


def _router(x_flat, gate_w, expert_bias):
    T = x_flat.shape[0]
    bm = 256
    gwt = jnp.zeros((HIDDEN, LANES), jnp.float32).at[:, :N_ROUTED].set(gate_w.T)
    bias = jnp.zeros((1, LANES), jnp.float32).at[0, :N_ROUTED].set(expert_bias)
    idx, w = pl.pallas_call(
        _router_body,
        grid=(T // bm,),
        in_specs=[
            pl.BlockSpec((bm, HIDDEN), lambda i: (i, 0)),
            pl.BlockSpec((HIDDEN, LANES), lambda i: (0, 0)),
            pl.BlockSpec((1, LANES), lambda i: (0, 0)),
        ],
        out_specs=[
            pl.BlockSpec((bm, LANES), lambda i: (i, 0)),
            pl.BlockSpec((bm, LANES), lambda i: (i, 0)),
        ],
        out_shape=[
            jax.ShapeDtypeStruct((T, LANES), jnp.int32),
            jax.ShapeDtypeStruct((T, LANES), jnp.float32),
        ],
    )(x_flat, gwt, bias)
    return idx[:, :TOP_K], w[:, :TOP_K]


def _swiglu(x, Wg, Wu, Wd):
    return (jax.nn.silu(x @ Wg) * (x @ Wu)) @ Wd


def kernel(x, gate_w, expert_bias, shared_Wg, shared_Wu, shared_Wd,
           routed_Wg, routed_Wu, routed_Wd):
    Bc, Sc, H = x.shape
    T = Bc * Sc
    x_flat = x.reshape(T, H)

    idx2, w2 = _router(x_flat, gate_w, expert_bias)

    shared_out = _swiglu(x_flat, shared_Wg[0], shared_Wu[0], shared_Wd[0])

    out_flat = jnp.zeros((T, H), dtype=x.dtype)
    for e in range(N_ROUTED):
        w_e = jnp.sum(jnp.where(idx2 == e, w2, 0.0), axis=-1)
        y_e = _swiglu(x_flat, routed_Wg[e], routed_Wu[e], routed_Wd[e])
        out_flat = out_flat + w_e[:, None] * y_e

    output = (shared_out + out_flat).reshape(Bc, Sc, H)
    return output, idx2.reshape(Bc, Sc, TOP_K)


# speedup vs baseline: 1.1466x; 1.1466x over previous
import jax
import jax.numpy as jnp
from jax.experimental import pallas as pl
from jax.experimental.pallas import tpu as pltpu

HIDDEN = 2048
INTER = 1408
N_ROUTED = 8
TOP_K = 2
BM = 256


def _mm_body(be_ref, na_ref, xd_ref, wg_ref, wu_ref, wd_ref, wrow_ref, out_ref):
    g = pl.program_id(0)

    @pl.when(g < na_ref[0])
    def _():
        xb = xd_ref[...].astype(jnp.bfloat16)
        gp = jax.lax.dot_general(
            xb, wg_ref[0], (((1,), (0,)), ((), ())),
            preferred_element_type=jnp.float32)
        up = jax.lax.dot_general(
            xb, wu_ref[0], (((1,), (0,)), ((), ())),
            preferred_element_type=jnp.float32)
        h = (jax.nn.silu(gp) * up).astype(jnp.bfloat16)
        y = jax.lax.dot_general(
            h, wd_ref[0], (((1,), (0,)), ((), ())),
            preferred_element_type=jnp.float32)
        out_ref[...] = y * wrow_ref[...]


def _grouped_ffn(xd, block_e, num_active, Wg, Wu, Wd, wrow, nb):
    R = xd.shape[0]

    def clamp(g, na_ref):
        return jnp.minimum(g, na_ref[0] - 1)

    grid_spec = pltpu.PrefetchScalarGridSpec(
        num_scalar_prefetch=2,
        grid=(nb,),
        in_specs=[
            pl.BlockSpec((BM, HIDDEN), lambda g, be, na: (clamp(g, na), 0)),
            pl.BlockSpec((1, HIDDEN, INTER),
                         lambda g, be, na: (be[clamp(g, na)], 0, 0)),
            pl.BlockSpec((1, HIDDEN, INTER),
                         lambda g, be, na: (be[clamp(g, na)], 0, 0)),
            pl.BlockSpec((1, INTER, HIDDEN),
                         lambda g, be, na: (be[clamp(g, na)], 0, 0)),
            pl.BlockSpec((BM, 1), lambda g, be, na: (clamp(g, na), 0)),
        ],
        out_specs=pl.BlockSpec((BM, HIDDEN), lambda g, be, na: (clamp(g, na), 0)),
    )
    return pl.pallas_call(
        _mm_body,
        grid_spec=grid_spec,
        out_shape=jax.ShapeDtypeStruct((R, HIDDEN), jnp.float32),
    )(block_e, num_active, xd, Wg, Wu, Wd, wrow)


def kernel(x, gate_w, expert_bias, shared_Wg, shared_Wu, shared_Wd,
           routed_Wg, routed_Wu, routed_Wd):
    Bc, Sc, H = x.shape
    T = Bc * Sc
    A = T * TOP_K
    x_flat = x.reshape(T, H)

    x_fp32 = x.astype(jnp.float32)
    gate_logits = x_fp32 @ gate_w.astype(jnp.float32).T
    scores = jax.nn.sigmoid(gate_logits)
    tie = jnp.arange(N_ROUTED, dtype=jnp.float32) * 1e-6
    scores_for_routing = scores + expert_bias.astype(jnp.float32) + tie
    _, top_k_indices = jax.lax.top_k(scores_for_routing, TOP_K)
    top_k_scores = jnp.take_along_axis(scores, top_k_indices, axis=-1)
    denom = jnp.maximum(jnp.sum(top_k_scores, axis=-1, keepdims=True), 1e-9)
    top_k_weights = (top_k_scores / denom).astype(x.dtype)

    idx2 = top_k_indices.reshape(T, TOP_K)
    w2 = top_k_weights.reshape(T, TOP_K)

    e_flat = idx2.reshape(A)
    w_flat = w2.reshape(A)
    tok_ids = (jnp.arange(A, dtype=jnp.int32) // TOP_K)
    onehot = (e_flat[:, None] == jnp.arange(N_ROUTED)[None, :]).astype(jnp.int32)
    counts = jnp.sum(onehot, axis=0)
    rank = jnp.sum(jnp.where(onehot == 1, jnp.cumsum(onehot, axis=0) - 1, 0),
                   axis=1)
    padded = ((counts + BM - 1) // BM) * BM
    offs = jnp.concatenate([jnp.zeros((1,), jnp.int32),
                            jnp.cumsum(padded).astype(jnp.int32)])
    pos = offs[e_flat] + rank

    MAXR = A + N_ROUTED * BM
    sorted_ids = jnp.zeros((MAXR,), jnp.int32).at[pos].set(tok_ids)
    sorted_w = jnp.zeros((MAXR,), jnp.float32).at[pos].set(w_flat)

    NB_SH = T // BM
    NB_RT = MAXR // BM
    NB = NB_SH + NB_RT
    block_start = jnp.arange(NB_RT, dtype=jnp.int32) * BM
    block_e = jnp.clip(
        jnp.sum(block_start[:, None] >= offs[None, 1:], axis=1), 0, N_ROUTED - 1
    ).astype(jnp.int32)
    full_be = jnp.concatenate(
        [jnp.full((NB_SH,), N_ROUTED, jnp.int32), block_e])
    num_active = (NB_SH + offs[N_ROUTED] // BM).reshape(1).astype(jnp.int32)

    full_ids = jnp.concatenate([jnp.arange(T, dtype=jnp.int32), sorted_ids])
    xd = jnp.take(x_flat, full_ids, axis=0)
    full_w = jnp.concatenate([jnp.ones((T,), jnp.float32), sorted_w])

    Wg9 = jnp.concatenate([routed_Wg, shared_Wg]).astype(jnp.bfloat16)
    Wu9 = jnp.concatenate([routed_Wu, shared_Wu]).astype(jnp.bfloat16)
    Wd9 = jnp.concatenate([routed_Wd, shared_Wd]).astype(jnp.bfloat16)
    y = _grouped_ffn(xd, full_be, num_active, Wg9, Wu9, Wd9,
                     full_w[:, None], NB)

    p = pos.reshape(T, TOP_K)
    out_flat = (y[:T]
                + jnp.take(y, T + p[:, 0], axis=0)
                + jnp.take(y, T + p[:, 1], axis=0))
    return out_flat.reshape(Bc, Sc, H), top_k_indices


# speedup vs baseline: 1.2536x; 1.0933x over previous
import jax
import jax.numpy as jnp
from jax.experimental import pallas as pl
from jax.experimental.pallas import tpu as pltpu

HIDDEN = 2048
INTER = 1408
N_ROUTED = 8
TOP_K = 2
BM = 256
NB_SH = 4096 // BM


def _ffn(xb, wg, wu, wd):
    gp = jax.lax.dot_general(xb, wg, (((1,), (0,)), ((), ())),
                             preferred_element_type=jnp.float32)
    up = jax.lax.dot_general(xb, wu, (((1,), (0,)), ((), ())),
                             preferred_element_type=jnp.float32)
    h = (jax.nn.silu(gp) * up).astype(jnp.bfloat16)
    return jax.lax.dot_general(h, wd, (((1,), (0,)), ((), ())),
                               preferred_element_type=jnp.float32)


def _mm_body(be_ref, na_ref, x_ref, xd_ref, wg_ref, wu_ref, wd_ref, out_ref):
    g = pl.program_id(0)

    @pl.when(g < NB_SH)
    def _():
        out_ref[...] = _ffn(x_ref[...].astype(jnp.bfloat16),
                            wg_ref[0], wu_ref[0], wd_ref[0])

    @pl.when((g >= NB_SH) & (g < na_ref[0]))
    def _():
        out_ref[...] = _ffn(xd_ref[...].astype(jnp.bfloat16),
                            wg_ref[0], wu_ref[0], wd_ref[0])


def _grouped_ffn(x_flat, xd, block_e, num_active, Wg, Wu, Wd, nb):
    T = x_flat.shape[0]
    R = xd.shape[0]
    nbr = R // BM

    def clampg(g, na_ref):
        return jnp.minimum(g, na_ref[0] - 1)

    grid_spec = pltpu.PrefetchScalarGridSpec(
        num_scalar_prefetch=2,
        grid=(nb,),
        in_specs=[
            pl.BlockSpec((BM, HIDDEN),
                         lambda g, be, na: (jnp.minimum(g, NB_SH - 1), 0)),
            pl.BlockSpec((BM, HIDDEN),
                         lambda g, be, na: (
                             jnp.clip(g - NB_SH, 0, nbr - 1), 0)),
            pl.BlockSpec((1, HIDDEN, INTER),
                         lambda g, be, na: (be[clampg(g, na)], 0, 0)),
            pl.BlockSpec((1, HIDDEN, INTER),
                         lambda g, be, na: (be[clampg(g, na)], 0, 0)),
            pl.BlockSpec((1, INTER, HIDDEN),
                         lambda g, be, na: (be[clampg(g, na)], 0, 0)),
        ],
        out_specs=pl.BlockSpec((BM, HIDDEN),
                               lambda g, be, na: (clampg(g, na), 0)),
    )
    return pl.pallas_call(
        _mm_body,
        grid_spec=grid_spec,
        out_shape=jax.ShapeDtypeStruct((T + R, HIDDEN), jnp.float32),
    )(block_e, num_active, x_flat, xd, Wg, Wu, Wd)


def kernel(x, gate_w, expert_bias, shared_Wg, shared_Wu, shared_Wd,
           routed_Wg, routed_Wu, routed_Wd):
    Bc, Sc, H = x.shape
    T = Bc * Sc
    A = T * TOP_K
    x_flat = x.reshape(T, H)

    x_fp32 = x.astype(jnp.float32)
    gate_logits = x_fp32 @ gate_w.astype(jnp.float32).T
    scores = jax.nn.sigmoid(gate_logits)
    tie = jnp.arange(N_ROUTED, dtype=jnp.float32) * 1e-6
    scores_for_routing = scores + expert_bias.astype(jnp.float32) + tie
    _, top_k_indices = jax.lax.top_k(scores_for_routing, TOP_K)
    top_k_scores = jnp.take_along_axis(scores, top_k_indices, axis=-1)
    denom = jnp.maximum(jnp.sum(top_k_scores, axis=-1, keepdims=True), 1e-9)
    top_k_weights = (top_k_scores / denom).astype(x.dtype)

    idx2 = top_k_indices.reshape(T, TOP_K)
    w2 = top_k_weights.reshape(T, TOP_K)

    e_flat = idx2.reshape(A)
    tok_ids = (jnp.arange(A, dtype=jnp.int32) // TOP_K)
    onehot = (e_flat[:, None] == jnp.arange(N_ROUTED)[None, :]).astype(jnp.int32)
    counts = jnp.sum(onehot, axis=0)
    rank = jnp.sum(jnp.where(onehot == 1, jnp.cumsum(onehot, axis=0) - 1, 0),
                   axis=1)
    padded = ((counts + BM - 1) // BM) * BM
    offs = jnp.concatenate([jnp.zeros((1,), jnp.int32),
                            jnp.cumsum(padded).astype(jnp.int32)])
    pos = offs[e_flat] + rank

    MAXR = A + N_ROUTED * BM
    sorted_ids = jnp.zeros((MAXR,), jnp.int32).at[pos].set(tok_ids)

    NB_RT = MAXR // BM
    NB = NB_SH + NB_RT
    block_start = jnp.arange(NB_RT, dtype=jnp.int32) * BM
    block_e = jnp.clip(
        jnp.sum(block_start[:, None] >= offs[None, 1:], axis=1), 0, N_ROUTED - 1
    ).astype(jnp.int32)
    full_be = jnp.concatenate(
        [jnp.full((NB_SH,), N_ROUTED, jnp.int32), block_e])
    num_active = (NB_SH + offs[N_ROUTED] // BM).reshape(1).astype(jnp.int32)

    xd = jnp.take(x_flat, sorted_ids, axis=0)

    Wg9 = jnp.concatenate([routed_Wg, shared_Wg]).astype(jnp.bfloat16)
    Wu9 = jnp.concatenate([routed_Wu, shared_Wu]).astype(jnp.bfloat16)
    Wd9 = jnp.concatenate([routed_Wd, shared_Wd]).astype(jnp.bfloat16)
    y = _grouped_ffn(x_flat, xd, full_be, num_active, Wg9, Wu9, Wd9, NB)

    p = pos.reshape(T, TOP_K)
    out_flat = (y[:T]
                + w2[:, 0:1] * jnp.take(y, T + p[:, 0], axis=0)
                + w2[:, 1:2] * jnp.take(y, T + p[:, 1], axis=0))
    return out_flat.reshape(Bc, Sc, H), top_k_indices
